# baseline (speedup 1.0000x reference)
import functools

import jax
import jax.numpy as jnp
from jax import lax
from jax.experimental import pallas as pl
from jax.experimental.pallas import tpu as pltpu
from jax.experimental.pallas import tpu_sc as plsc


def _table_body(embed_ref, ln_w_ref, ln_b_ref, w_ref, b_ref, table_ref):
    h = embed_ref[...]
    mean = jnp.mean(h, axis=1, keepdims=True)
    var = jnp.mean(jnp.square(h - mean), axis=1, keepdims=True)
    hn = (h - mean) / jnp.sqrt(var + 1e-5) * ln_w_ref[...] + ln_b_ref[...]
    table_ref[...] = (
        lax.dot_general(hn, w_ref[...], (((1,), (1,)), ((), ())),
                        preferred_element_type=jnp.float32)
        + b_ref[...]
    )


def _make_table(embed, ln_w, ln_b, W, b):
    V, E = embed.shape
    return pl.pallas_call(
        _table_body,
        out_shape=jax.ShapeDtypeStruct((V, V), jnp.float32),
    )(embed, ln_w.reshape(1, E), ln_b.reshape(1, E), W, b.reshape(1, V))


def _gather_body(n_chunks, chunk, table_hbm, idx_hbm, out_hbm, idx_v, rows_v, sem):
    info = plsc.get_sparse_core_info()
    nc, ns = info.num_cores, info.num_subcores
    wid = lax.axis_index("s") * nc + lax.axis_index("c")
    base = wid * (n_chunks * chunk)

    def body(c, _):
        off = base + c * chunk
        pltpu.sync_copy(idx_hbm.at[pl.ds(off, chunk)], idx_v)
        pltpu.async_copy(table_hbm.at[idx_v], rows_v, sem).wait()
        pltpu.sync_copy(rows_v, out_hbm.at[pl.ds(off, chunk)])
        return 0

    lax.fori_loop(0, n_chunks, body, 0)


def _gather_rows(table, ids):
    V, D = table.shape
    (B,) = ids.shape
    info = plsc.get_sparse_core_info()
    nw = info.num_cores * info.num_subcores
    chunk = 128
    n_chunks = B // (nw * chunk)
    mesh = plsc.VectorSubcoreMesh(core_axis_name="c", subcore_axis_name="s")
    grab = functools.partial(
        pl.kernel,
        mesh=mesh,
        out_type=jax.ShapeDtypeStruct((B, D), jnp.float32),
        scratch_types=[
            pltpu.VMEM((chunk,), jnp.int32),
            pltpu.VMEM((chunk, D), jnp.float32),
            pltpu.SemaphoreType.DMA,
        ],
        compiler_params=pltpu.CompilerParams(use_tc_tiling_on_sc=False),
    )(functools.partial(_gather_body, n_chunks, chunk))
    return grab(table, ids)


def kernel(input_ids, embed, ln_w, ln_b, W, b):
    Bt, Lt = input_ids.shape
    V, _ = embed.shape
    table = _make_table(embed, ln_w, ln_b, W, b)
    ids = input_ids.reshape(-1).astype(jnp.int32)
    out = _gather_rows(table, ids)
    return out.reshape(Bt, Lt, V)


# speedup vs baseline: 1.0817x; 1.0817x over previous
import functools

import jax
import jax.numpy as jnp
from jax import lax
from jax.experimental import pallas as pl
from jax.experimental.pallas import tpu as pltpu
from jax.experimental.pallas import tpu_sc as plsc


def _table_body(embed_ref, ln_w_ref, ln_b_ref, w_ref, b_ref, table_ref):
    h = embed_ref[...]
    mean = jnp.mean(h, axis=1, keepdims=True)
    var = jnp.mean(jnp.square(h - mean), axis=1, keepdims=True)
    hn = (h - mean) / jnp.sqrt(var + 1e-5) * ln_w_ref[...] + ln_b_ref[...]
    table_ref[...] = (
        lax.dot_general(hn, w_ref[...], (((1,), (1,)), ((), ())),
                        preferred_element_type=jnp.float32)
        + b_ref[...]
    )


def _make_table(embed, ln_w, ln_b, W, b):
    V, E = embed.shape
    return pl.pallas_call(
        _table_body,
        out_shape=jax.ShapeDtypeStruct((V, V), jnp.float32),
    )(embed, ln_w.reshape(1, E), ln_b.reshape(1, E), W, b.reshape(1, V))


def _gather_body(n_pairs, chunk, table_hbm, idx_hbm, out_hbm,
                 idx_v, rows0, rows1, table_sh, sem0, sem1):
    info = plsc.get_sparse_core_info()
    nc = info.num_cores
    sid = lax.axis_index("s")
    wid = sid * nc + lax.axis_index("c")
    per_tile = 2 * n_pairs * chunk
    base = wid * per_tile

    @pl.when(sid == 0)
    def _stage():
        pltpu.sync_copy(table_hbm, table_sh)

    plsc.subcore_barrier()
    pltpu.sync_copy(idx_hbm.at[pl.ds(base, per_tile)], idx_v)

    def step(i, _):
        off0 = (2 * i) * chunk
        off1 = off0 + chunk

        @pl.when(i > 0)
        def _reclaim():
            pltpu.make_async_copy(
                rows0, out_hbm.at[pl.ds(base + off0, chunk)], sem0).wait()
            pltpu.make_async_copy(
                rows1, out_hbm.at[pl.ds(base + off1, chunk)], sem1).wait()

        pltpu.sync_copy(table_sh.at[idx_v.at[pl.ds(off0, chunk)]], rows0)
        pltpu.async_copy(rows0, out_hbm.at[pl.ds(base + off0, chunk)], sem0)
        pltpu.sync_copy(table_sh.at[idx_v.at[pl.ds(off1, chunk)]], rows1)
        pltpu.async_copy(rows1, out_hbm.at[pl.ds(base + off1, chunk)], sem1)
        return 0

    lax.fori_loop(0, n_pairs, step, 0)
    pltpu.make_async_copy(rows0, out_hbm.at[pl.ds(base, chunk)], sem0).wait()
    pltpu.make_async_copy(rows1, out_hbm.at[pl.ds(base, chunk)], sem1).wait()


def _gather_rows(table, ids):
    V, D = table.shape
    (B,) = ids.shape
    info = plsc.get_sparse_core_info()
    nw = info.num_cores * info.num_subcores
    per_tile = B // nw
    chunk = 16
    n_pairs = per_tile // (2 * chunk)
    mesh = plsc.VectorSubcoreMesh(core_axis_name="c", subcore_axis_name="s")
    grab = functools.partial(
        pl.kernel,
        mesh=mesh,
        out_type=jax.ShapeDtypeStruct((B, D), jnp.float32),
        scratch_types=[
            pltpu.VMEM((per_tile,), jnp.int32),
            pltpu.VMEM((chunk, D), jnp.float32),
            pltpu.VMEM((chunk, D), jnp.float32),
            pltpu.VMEM_SHARED((V, D), jnp.float32),
            pltpu.SemaphoreType.DMA,
            pltpu.SemaphoreType.DMA,
        ],
        compiler_params=pltpu.CompilerParams(use_tc_tiling_on_sc=False),
    )(functools.partial(_gather_body, n_pairs, chunk))
    return grab(table, ids)


def kernel(input_ids, embed, ln_w, ln_b, W, b):
    Bt, Lt = input_ids.shape
    V, _ = embed.shape
    table = _make_table(embed, ln_w, ln_b, W, b)
    ids = input_ids.reshape(-1).astype(jnp.int32)
    out = _gather_rows(table, ids)
    return out.reshape(Bt, Lt, V)
